# baseline (speedup 1.0000x reference)
import functools

import jax
import jax.numpy as jnp
from jax import lax
from jax.experimental import pallas as pl
from jax.experimental.pallas import tpu as pltpu
from jax.experimental.pallas import tpu_sc as plsc

_MARGIN = 0.3

_B, _C, _D = 4096, 64, 64
_NC, _NS = 2, 16
_NW = _NC * _NS
_BW = _B // _NW
_NG = _BW // 16


def _sc_rej_body(wf_hbm, labels_hbm, out_hbm, wfbuf, labbuf, acc):
    wid = lax.axis_index("s") * _NC + lax.axis_index("c")
    base = wid * _BW

    pltpu.sync_copy(labels_hbm.at[pl.ds(base, _BW)], labbuf)
    for g in range(_NG):
        acc[pl.ds(g * 16, 16)] = jnp.zeros((16,), jnp.float32)

    def cls_body(c, carry):
        pltpu.sync_copy(wf_hbm.at[c].at[pl.ds(base, _BW)], wfbuf)
        for g in range(_NG):
            rows = g * 16 + lax.iota(jnp.int32, 16)
            m = plsc.load_gather(wfbuf, [rows, jnp.zeros((16,), jnp.int32)])
            for j in range(1, _D):
                col = jnp.full((16,), j, jnp.int32)
                m = jnp.maximum(m, plsc.load_gather(wfbuf, [rows, col]))
            p = 1.0 / (1.0 + jnp.exp(-m))
            r = jnp.maximum(p - _MARGIN, 0.0)
            lab = plsc.load_gather(labbuf, [rows, jnp.full((16,), c, jnp.int32)])
            r = jnp.where(lab == 0.0, r, 0.0)
            acc[pl.ds(g * 16, 16)] += r
        return carry

    lax.fori_loop(0, _C, cls_body, 0)
    pltpu.sync_copy(acc, out_hbm.at[pl.ds(base, _BW)])


@functools.partial(
    pl.kernel,
    out_type=jax.ShapeDtypeStruct((_B,), jnp.float32),
    mesh=plsc.VectorSubcoreMesh(core_axis_name="c", subcore_axis_name="s"),
    scratch_types=[
        pltpu.VMEM((_BW, _D), jnp.float32),
        pltpu.VMEM((_BW, _C), jnp.float32),
        pltpu.VMEM((_BW,), jnp.float32),
    ],
    compiler_params=pltpu.CompilerParams(needs_layout_passes=False),
)
def _sc_rej(wf_hbm, labels_hbm, out_hbm, wfbuf, labbuf, acc):
    _sc_rej_body(wf_hbm, labels_hbm, out_hbm, wfbuf, labbuf, acc)


def _bce_body(logits_ref, labels_ref, out_ref):
    logits = logits_ref[...]
    labels = labels_ref[...]
    bce = jnp.maximum(logits, 0.0) - logits * labels + jnp.log1p(
        jnp.exp(-jnp.abs(logits)))
    out_ref[...] = jnp.sum(bce, axis=1).reshape(1, 1, -1)


def kernel(logits, wf, labels):
    B, C = logits.shape

    rej = _sc_rej(wf, labels)

    _BBLK = 1024
    bce = pl.pallas_call(
        _bce_body,
        grid=(B // _BBLK,),
        in_specs=[
            pl.BlockSpec((_BBLK, C), lambda i: (i, 0)),
            pl.BlockSpec((_BBLK, C), lambda i: (i, 0)),
        ],
        out_specs=pl.BlockSpec((1, 1, _BBLK), lambda i: (i, 0, 0)),
        out_shape=jax.ShapeDtypeStruct((B // _BBLK, 1, _BBLK), jnp.float32),
    )(logits, labels)

    return rej + bce.reshape(B)


# speedup vs baseline: 1.0940x; 1.0940x over previous
import functools

import jax
import jax.numpy as jnp
from jax import lax
from jax.experimental import pallas as pl
from jax.experimental.pallas import tpu as pltpu
from jax.experimental.pallas import tpu_sc as plsc

_MARGIN = 0.3

_B, _C, _D = 4096, 64, 64
_NC, _NS = 2, 16
_NW = _NC * _NS
_BW = _B // _NW
_NG = _BW // 16


_NBUF = 4


def _sc_rej_body(wf_hbm, labels_hbm, out_hbm, wfbuf, labbuf, acc, sems):
    wid = lax.axis_index("s") * _NC + lax.axis_index("c")
    base = wid * _BW

    pltpu.sync_copy(labels_hbm.at[pl.ds(base, _BW)], labbuf)
    for g in range(_NG):
        acc[pl.ds(g * 16, 16)] = jnp.zeros((16,), jnp.float32)

    def dma(c, u):
        return pltpu.make_async_copy(
            wf_hbm.at[c].at[pl.ds(base, _BW)], wfbuf.at[u], sems[u])

    for u in range(_NBUF):
        dma(u, u).start()

    def compute_one(c, u):
        for g in range(_NG):
            rows = g * 16 + lax.iota(jnp.int32, 16)
            buf2d = wfbuf.at[u]
            m = plsc.load_gather(buf2d, [rows, jnp.zeros((16,), jnp.int32)])
            for j in range(1, _D):
                col = jnp.full((16,), j, jnp.int32)
                m = jnp.maximum(m, plsc.load_gather(buf2d, [rows, col]))
            p = 1.0 / (1.0 + jnp.exp(-m))
            r = jnp.maximum(p - _MARGIN, 0.0)
            lab = plsc.load_gather(labbuf, [rows, jnp.full((16,), c, jnp.int32)])
            r = jnp.where(lab == 0.0, r, 0.0)
            acc[pl.ds(g * 16, 16)] += r

    def block_body(k, carry):
        for u in range(_NBUF):
            c = _NBUF * k + u
            dma(c, u).wait()

            @pl.when(c + _NBUF < _C)
            def _prefetch():
                dma(c + _NBUF, u).start()

            compute_one(c, u)
        return carry

    lax.fori_loop(0, _C // _NBUF, block_body, 0)
    pltpu.sync_copy(acc, out_hbm.at[pl.ds(base, _BW)])


@functools.partial(
    pl.kernel,
    out_type=jax.ShapeDtypeStruct((_B,), jnp.float32),
    mesh=plsc.VectorSubcoreMesh(core_axis_name="c", subcore_axis_name="s"),
    scratch_types=[
        pltpu.VMEM((_NBUF, _BW, _D), jnp.float32),
        pltpu.VMEM((_BW, _C), jnp.float32),
        pltpu.VMEM((_BW,), jnp.float32),
        [pltpu.SemaphoreType.DMA] * _NBUF,
    ],
    compiler_params=pltpu.CompilerParams(needs_layout_passes=False),
)
def _sc_rej(wf_hbm, labels_hbm, out_hbm, wfbuf, labbuf, acc, sems):
    _sc_rej_body(wf_hbm, labels_hbm, out_hbm, wfbuf, labbuf, acc, sems)


def _bce_body(logits_ref, labels_ref, out_ref):
    logits = logits_ref[...]
    labels = labels_ref[...]
    bce = jnp.maximum(logits, 0.0) - logits * labels + jnp.log1p(
        jnp.exp(-jnp.abs(logits)))
    out_ref[...] = jnp.sum(bce, axis=1).reshape(1, 1, -1)


def kernel(logits, wf, labels):
    B, C = logits.shape

    rej = _sc_rej(wf, labels)

    _BBLK = 1024
    bce = pl.pallas_call(
        _bce_body,
        grid=(B // _BBLK,),
        in_specs=[
            pl.BlockSpec((_BBLK, C), lambda i: (i, 0)),
            pl.BlockSpec((_BBLK, C), lambda i: (i, 0)),
        ],
        out_specs=pl.BlockSpec((1, 1, _BBLK), lambda i: (i, 0, 0)),
        out_shape=jax.ShapeDtypeStruct((B // _BBLK, 1, _BBLK), jnp.float32),
    )(logits, labels)

    return rej + bce.reshape(B)


# speedup vs baseline: 1.6736x; 1.5298x over previous
import functools

import jax
import jax.numpy as jnp
from jax import lax
from jax.experimental import pallas as pl
from jax.experimental.pallas import tpu as pltpu
from jax.experimental.pallas import tpu_sc as plsc

_MARGIN = 0.3

_B, _C, _D = 4096, 64, 64
_NC, _NS = 2, 16
_NW = _NC * _NS
_BW = _B // _NW
_NG = _BW // 16


_NBUF = 4


def _sc_rej_body(wf_hbm, labels_hbm, out_hbm, wfbuf, labbuf, acc, sems):
    wid = lax.axis_index("s") * _NC + lax.axis_index("c")
    base = wid * _BW

    pltpu.sync_copy(labels_hbm.at[pl.ds(base, _BW)], labbuf)
    for g in range(_NG):
        acc[pl.ds(g * 16, 16)] = jnp.zeros((16,), jnp.float32)

    def dma(c, u):
        return pltpu.make_async_copy(
            wf_hbm.at[c].at[pl.ds(base, _BW)], wfbuf.at[u], sems[u])

    for u in range(_NBUF):
        dma(u, u).start()

    def compute_one(c, u):
        lane = lax.iota(jnp.int32, 16)
        for g in range(_NG):
            rows = g * 16 + lane
            buf2d = wfbuf.at[u]
            m = plsc.load_gather(buf2d, [rows, lane])
            for j in range(1, _D):
                col = (lane + j) & (_D - 1)
                m = jnp.maximum(m, plsc.load_gather(buf2d, [rows, col]))
            p = 1.0 / (1.0 + jnp.exp(-m))
            r = jnp.maximum(p - _MARGIN, 0.0)
            lab = plsc.load_gather(labbuf, [rows, jnp.full((16,), c, jnp.int32)])
            r = jnp.where(lab == 0.0, r, 0.0)
            acc[pl.ds(g * 16, 16)] += r

    def block_body(k, carry):
        for u in range(_NBUF):
            c = _NBUF * k + u
            dma(c, u).wait()
            compute_one(c, u)

            @pl.when(c + _NBUF < _C)
            def _prefetch():
                dma(c + _NBUF, u).start()
        return carry

    lax.fori_loop(0, _C // _NBUF, block_body, 0)
    pltpu.sync_copy(acc, out_hbm.at[pl.ds(base, _BW)])


@functools.partial(
    pl.kernel,
    out_type=jax.ShapeDtypeStruct((_B,), jnp.float32),
    mesh=plsc.VectorSubcoreMesh(core_axis_name="c", subcore_axis_name="s"),
    scratch_types=[
        pltpu.VMEM((_NBUF, _BW, _D), jnp.float32),
        pltpu.VMEM((_BW, _C), jnp.float32),
        pltpu.VMEM((_BW,), jnp.float32),
        [pltpu.SemaphoreType.DMA] * _NBUF,
    ],
    compiler_params=pltpu.CompilerParams(needs_layout_passes=False),
)
def _sc_rej(wf_hbm, labels_hbm, out_hbm, wfbuf, labbuf, acc, sems):
    _sc_rej_body(wf_hbm, labels_hbm, out_hbm, wfbuf, labbuf, acc, sems)


def _bce_body(logits_ref, labels_ref, out_ref):
    logits = logits_ref[...]
    labels = labels_ref[...]
    bce = jnp.maximum(logits, 0.0) - logits * labels + jnp.log1p(
        jnp.exp(-jnp.abs(logits)))
    out_ref[...] = jnp.sum(bce, axis=1).reshape(1, 1, -1)


def kernel(logits, wf, labels):
    B, C = logits.shape

    rej = _sc_rej(wf, labels)

    _BBLK = 1024
    bce = pl.pallas_call(
        _bce_body,
        grid=(B // _BBLK,),
        in_specs=[
            pl.BlockSpec((_BBLK, C), lambda i: (i, 0)),
            pl.BlockSpec((_BBLK, C), lambda i: (i, 0)),
        ],
        out_specs=pl.BlockSpec((1, 1, _BBLK), lambda i: (i, 0, 0)),
        out_shape=jax.ShapeDtypeStruct((B // _BBLK, 1, _BBLK), jnp.float32),
    )(logits, labels)

    return rej + bce.reshape(B)


# speedup vs baseline: 1.6892x; 1.0093x over previous
import functools

import jax
import jax.numpy as jnp
from jax import lax
from jax.experimental import pallas as pl
from jax.experimental.pallas import tpu as pltpu
from jax.experimental.pallas import tpu_sc as plsc

_MARGIN = 0.3

_B, _C, _D = 4096, 64, 64
_NC, _NS = 2, 16
_NW = _NC * _NS
_BW = _B // _NW
_NG = _BW // 16


_NBUF = 2
_CH = 2
_NCHUNK = _C // _CH


def _sc_rej_body(wf_hbm, labels_hbm, out_hbm, wfbuf, labbuf, acc, sems):
    wid = lax.axis_index("s") * _NC + lax.axis_index("c")
    base = wid * _BW

    pltpu.sync_copy(labels_hbm.at[pl.ds(base, _BW)], labbuf)
    for g in range(_NG):
        acc[pl.ds(g * 16, 16)] = jnp.zeros((16,), jnp.float32)

    def dma(ch, u):
        return pltpu.make_async_copy(
            wf_hbm.at[pl.ds(ch * _CH, _CH), pl.ds(base, _BW)],
            wfbuf.at[u], sems[u])

    for u in range(_NBUF):
        dma(u, u).start()

    def compute_one(c, u, cc):
        lane = lax.iota(jnp.int32, 16)
        for g in range(_NG):
            rows = g * 16 + lane
            buf2d = wfbuf.at[u, cc]
            m = plsc.load_gather(buf2d, [rows, lane])
            for j in range(1, _D):
                col = (lane + j) & (_D - 1)
                m = jnp.maximum(m, plsc.load_gather(buf2d, [rows, col]))
            p = 1.0 / (1.0 + jnp.exp(-m))
            r = jnp.maximum(p - _MARGIN, 0.0)
            lab = plsc.load_gather(labbuf, [rows, jnp.full((16,), c, jnp.int32)])
            r = jnp.where(lab == 0.0, r, 0.0)
            acc[pl.ds(g * 16, 16)] += r

    def block_body(k, carry):
        for u in range(_NBUF):
            ch = _NBUF * k + u
            dma(ch, u).wait()
            for cc in range(_CH):
                compute_one(ch * _CH + cc, u, cc)

            @pl.when(ch + _NBUF < _NCHUNK)
            def _prefetch():
                dma(ch + _NBUF, u).start()
        return carry

    lax.fori_loop(0, _NCHUNK // _NBUF, block_body, 0)
    pltpu.sync_copy(acc, out_hbm.at[pl.ds(base, _BW)])


@functools.partial(
    pl.kernel,
    out_type=jax.ShapeDtypeStruct((_B,), jnp.float32),
    mesh=plsc.VectorSubcoreMesh(core_axis_name="c", subcore_axis_name="s"),
    scratch_types=[
        pltpu.VMEM((_NBUF, _CH, _BW, _D), jnp.float32),
        pltpu.VMEM((_BW, _C), jnp.float32),
        pltpu.VMEM((_BW,), jnp.float32),
        [pltpu.SemaphoreType.DMA] * _NBUF,
    ],
    compiler_params=pltpu.CompilerParams(needs_layout_passes=False),
)
def _sc_rej(wf_hbm, labels_hbm, out_hbm, wfbuf, labbuf, acc, sems):
    _sc_rej_body(wf_hbm, labels_hbm, out_hbm, wfbuf, labbuf, acc, sems)


def _bce_body(logits_ref, labels_ref, out_ref):
    logits = logits_ref[...]
    labels = labels_ref[...]
    bce = jnp.maximum(logits, 0.0) - logits * labels + jnp.log1p(
        jnp.exp(-jnp.abs(logits)))
    out_ref[...] = jnp.sum(bce, axis=1).reshape(1, 1, -1)


def kernel(logits, wf, labels):
    B, C = logits.shape

    rej = _sc_rej(wf, labels)

    _BBLK = 1024
    bce = pl.pallas_call(
        _bce_body,
        grid=(B // _BBLK,),
        in_specs=[
            pl.BlockSpec((_BBLK, C), lambda i: (i, 0)),
            pl.BlockSpec((_BBLK, C), lambda i: (i, 0)),
        ],
        out_specs=pl.BlockSpec((1, 1, _BBLK), lambda i: (i, 0, 0)),
        out_shape=jax.ShapeDtypeStruct((B // _BBLK, 1, _BBLK), jnp.float32),
    )(logits, labels)

    return rej + bce.reshape(B)


# speedup vs baseline: 2.2414x; 1.3269x over previous
import functools

import jax
import jax.numpy as jnp
from jax import lax
from jax.experimental import pallas as pl
from jax.experimental.pallas import tpu as pltpu
from jax.experimental.pallas import tpu_sc as plsc

_MARGIN = 0.3

_B, _C, _D = 4096, 64, 64
_NC, _NS = 2, 16
_NW = _NC * _NS
_BW = _B // _NW
_NG = _BW // 16

_CSC = 32

_NBUF = 2
_CH = 2
_NCHUNK = _CSC // _CH


def _sc_rej_body(wf_hbm, labels_hbm, out_hbm, wfbuf, labbuf, acc, sems):
    wid = lax.axis_index("s") * _NC + lax.axis_index("c")
    base = wid * _BW

    pltpu.sync_copy(labels_hbm.at[pl.ds(base, _BW)], labbuf)
    for g in range(_NG):
        acc[pl.ds(g * 16, 16)] = jnp.zeros((16,), jnp.float32)

    def dma(ch, u):
        return pltpu.make_async_copy(
            wf_hbm.at[pl.ds(ch * _CH, _CH), pl.ds(base, _BW)],
            wfbuf.at[u], sems[u])

    for u in range(_NBUF):
        dma(u, u).start()

    def compute_one(c, u, cc):
        lane = lax.iota(jnp.int32, 16)
        for g in range(_NG):
            rows = g * 16 + lane
            buf2d = wfbuf.at[u, cc]
            m = plsc.load_gather(buf2d, [rows, lane])
            for j in range(1, _D):
                col = (lane + j) & (_D - 1)
                m = jnp.maximum(m, plsc.load_gather(buf2d, [rows, col]))
            p = 1.0 / (1.0 + jnp.exp(-m))
            r = jnp.maximum(p - _MARGIN, 0.0)
            lab = plsc.load_gather(labbuf, [rows, jnp.full((16,), c, jnp.int32)])
            r = jnp.where(lab == 0.0, r, 0.0)
            acc[pl.ds(g * 16, 16)] += r

    def block_body(k, carry):
        for u in range(_NBUF):
            ch = _NBUF * k + u
            dma(ch, u).wait()
            for cc in range(_CH):
                compute_one(ch * _CH + cc, u, cc)

            @pl.when(ch + _NBUF < _NCHUNK)
            def _prefetch():
                dma(ch + _NBUF, u).start()
        return carry

    lax.fori_loop(0, _NCHUNK // _NBUF, block_body, 0)
    pltpu.sync_copy(acc, out_hbm.at[pl.ds(base, _BW)])


@functools.partial(
    pl.kernel,
    out_type=jax.ShapeDtypeStruct((_B,), jnp.float32),
    mesh=plsc.VectorSubcoreMesh(core_axis_name="c", subcore_axis_name="s"),
    scratch_types=[
        pltpu.VMEM((_NBUF, _CH, _BW, _D), jnp.float32),
        pltpu.VMEM((_BW, _C), jnp.float32),
        pltpu.VMEM((_BW,), jnp.float32),
        [pltpu.SemaphoreType.DMA] * _NBUF,
    ],
    compiler_params=pltpu.CompilerParams(needs_layout_passes=False),
)
def _sc_rej(wf_hbm, labels_hbm, out_hbm, wfbuf, labbuf, acc, sems):
    _sc_rej_body(wf_hbm, labels_hbm, out_hbm, wfbuf, labbuf, acc, sems)


_BBLK = 512
_CBLK = 8


def _tc_rej_body(labels_t_ref, wf_ref, out_ref):
    j = pl.program_id(1)
    wfb = wf_ref[...]
    max_sim = jnp.max(wfb, axis=2)
    rej = jnp.maximum(jax.nn.sigmoid(max_sim) - _MARGIN, 0.0)
    mask = (labels_t_ref[...] == 0.0).astype(jnp.float32)
    part = jnp.sum(rej * mask, axis=0, keepdims=True)[None]

    @pl.when(j == 0)
    def _init():
        out_ref[...] = part

    @pl.when(j > 0)
    def _acc():
        out_ref[...] += part


def _bce_body(logits_ref, labels_ref, out_ref):
    logits = logits_ref[...]
    labels = labels_ref[...]
    bce = jnp.maximum(logits, 0.0) - logits * labels + jnp.log1p(
        jnp.exp(-jnp.abs(logits)))
    out_ref[...] = jnp.sum(bce, axis=1).reshape(1, 1, -1)


def kernel(logits, wf, labels):
    B, C = logits.shape
    D = wf.shape[2]
    labels_t = labels.T.reshape(C, B)
    coff = _CSC // _CBLK

    rej_sc = _sc_rej(wf, labels)

    rej_tc = pl.pallas_call(
        _tc_rej_body,
        grid=(B // _BBLK, (C - _CSC) // _CBLK),
        in_specs=[
            pl.BlockSpec((_CBLK, _BBLK), lambda i, j: (coff + j, i)),
            pl.BlockSpec((_CBLK, _BBLK, D), lambda i, j: (coff + j, i, 0)),
        ],
        out_specs=pl.BlockSpec((1, 1, _BBLK), lambda i, j: (i, 0, 0)),
        out_shape=jax.ShapeDtypeStruct((B // _BBLK, 1, _BBLK), jnp.float32),
    )(labels_t, wf)

    _BB = 1024
    bce = pl.pallas_call(
        _bce_body,
        grid=(B // _BB,),
        in_specs=[
            pl.BlockSpec((_BB, C), lambda i: (i, 0)),
            pl.BlockSpec((_BB, C), lambda i: (i, 0)),
        ],
        out_specs=pl.BlockSpec((1, 1, _BB), lambda i: (i, 0, 0)),
        out_shape=jax.ShapeDtypeStruct((B // _BB, 1, _BB), jnp.float32),
    )(logits, labels)

    return rej_sc + rej_tc.reshape(B) + bce.reshape(B)
